# Initial kernel scaffold; baseline (speedup 1.0000x reference)
import jax
import jax.numpy as jnp
from jax.experimental import pallas as pl


def kernel(x, table):
    raise NotImplementedError("write your pallas kernel here")


# speedup vs baseline: 67.2960x; 67.2960x over previous
import functools

import jax
import jax.numpy as jnp
from jax import lax
from jax.experimental import pallas as pl
from jax.experimental.pallas import tpu as pltpu
from jax.experimental.pallas import tpu_sc as plsc


def _emb_lookup(table, idx, n_per_w, chunk, num_cores):
    n, = idx.shape
    _, d = table.shape
    n_chunks = n_per_w // chunk
    mesh = plsc.VectorSubcoreMesh(core_axis_name="c", subcore_axis_name="s")

    @functools.partial(
        pl.kernel,
        mesh=mesh,
        out_type=jax.ShapeDtypeStruct((n, d), jnp.float32),
        compiler_params=pltpu.CompilerParams(use_tc_tiling_on_sc=False),
        scratch_types=[
            pltpu.VMEM((chunk,), jnp.int32),
            pltpu.VMEM((chunk, d), jnp.float32),
            pltpu.SemaphoreType.DMA,
        ],
    )
    def emb(table_hbm, idx_hbm, out_hbm, idx_v, rows_v, sem):
        wid = lax.axis_index("s") * num_cores + lax.axis_index("c")
        base = wid * n_per_w

        def body(i, carry):
            off = base + i * chunk
            pltpu.sync_copy(idx_hbm.at[pl.ds(off, chunk)], idx_v)
            pltpu.async_copy(table_hbm.at[idx_v], rows_v, sem).wait()
            pltpu.sync_copy(rows_v, out_hbm.at[pl.ds(off, chunk)])
            return carry

        lax.fori_loop(0, n_chunks, body, 0)

    return emb(table, idx)


def kernel(x, table):
    b, l = x.shape
    v, d = table.shape
    n = b * l
    idx = x.reshape(n)

    info = plsc.get_sparse_core_info()
    num_workers = info.num_cores * info.num_subcores
    n_per_w = n // num_workers
    chunk = 4096

    out = _emb_lookup(table, idx, n_per_w, chunk, info.num_cores)
    return out.reshape(b, l * d)


# speedup vs baseline: 69.9569x; 1.0395x over previous
import functools

import jax
import jax.numpy as jnp
from jax import lax
from jax.experimental import pallas as pl
from jax.experimental.pallas import tpu as pltpu
from jax.experimental.pallas import tpu_sc as plsc


def _emb_lookup(table, idx, n_per_w, chunk, num_cores):
    n, = idx.shape
    _, d = table.shape
    n_chunks = n_per_w // chunk
    assert n_chunks % 2 == 0 and n_chunks >= 4
    n_pairs = n_chunks // 2
    mesh = plsc.VectorSubcoreMesh(core_axis_name="c", subcore_axis_name="s")

    @functools.partial(
        pl.kernel,
        mesh=mesh,
        out_type=jax.ShapeDtypeStruct((n, d), jnp.float32),
        compiler_params=pltpu.CompilerParams(use_tc_tiling_on_sc=False),
        scratch_types=[
            pltpu.VMEM((chunk,), jnp.int32),
            pltpu.VMEM((chunk,), jnp.int32),
            pltpu.VMEM((chunk, d), jnp.float32),
            pltpu.VMEM((chunk, d), jnp.float32),
            pltpu.SemaphoreType.DMA,
            pltpu.SemaphoreType.DMA,
            pltpu.SemaphoreType.DMA,
            pltpu.SemaphoreType.DMA,
            pltpu.SemaphoreType.DMA,
            pltpu.SemaphoreType.DMA,
        ],
    )
    def emb(table_hbm, idx_hbm, out_hbm, idx0, idx1, rows0, rows1,
            si0, si1, sg0, sg1, so0, so1):
        wid = lax.axis_index("s") * num_cores + lax.axis_index("c")
        base = wid * n_per_w

        def start_idx(i, idx_v, si):
            pltpu.async_copy(idx_hbm.at[pl.ds(base + i * chunk, chunk)],
                             idx_v, si)

        def wait_idx(idx_v, si):
            pltpu.make_async_copy(idx_hbm.at[pl.ds(base, chunk)],
                                  idx_v, si).wait()

        def wait_out(rows_v, so):
            pltpu.make_async_copy(rows_v, out_hbm.at[pl.ds(base, chunk)],
                                  so).wait()

        def step(g, i, idx_v, rows_v, si, sg, so):
            @pl.when(g > 0)
            def _():
                wait_out(rows_v, so)

            wait_idx(idx_v, si)
            pltpu.async_copy(table_hbm.at[idx_v], rows_v, sg)
            pltpu.make_async_copy(table_hbm.at[idx_v], rows_v, sg).wait()
            pltpu.async_copy(rows_v,
                             out_hbm.at[pl.ds(base + i * chunk, chunk)], so)

            @pl.when(g < n_pairs - 1)
            def _():
                start_idx(i + 2, idx_v, si)

        start_idx(0, idx0, si0)
        start_idx(1, idx1, si1)

        def body(g, carry):
            step(g, 2 * g, idx0, rows0, si0, sg0, so0)
            step(g, 2 * g + 1, idx1, rows1, si1, sg1, so1)
            return carry

        lax.fori_loop(0, n_pairs, body, 0)

        wait_out(rows0, so0)
        wait_out(rows1, so1)

    return emb(table, idx)


def kernel(x, table):
    b, l = x.shape
    v, d = table.shape
    n = b * l
    idx = x.reshape(n)

    info = plsc.get_sparse_core_info()
    num_workers = info.num_cores * info.num_subcores
    n_per_w = n // num_workers
    chunk = 6400

    out = _emb_lookup(table, idx, n_per_w, chunk, info.num_cores)
    return out.reshape(b, l * d)


# speedup vs baseline: 76.8655x; 1.0988x over previous
import functools

import jax
import jax.numpy as jnp
from jax import lax
from jax.experimental import pallas as pl
from jax.experimental.pallas import tpu as pltpu
from jax.experimental.pallas import tpu_sc as plsc

_LANES = 128
_SUBL = 8
_TW = _LANES * _SUBL


def _emb_lookup(table, xq, n_tiles, tiles_per_w, bt_tiles, num_cores):
    v, d = table.shape
    n_out = xq.shape[0] * d
    mesh = plsc.VectorSubcoreMesh(core_axis_name="c", subcore_axis_name="s")

    @functools.partial(
        pl.kernel,
        mesh=mesh,
        out_type=jax.ShapeDtypeStruct((n_out,), jnp.float32),
        compiler_params=pltpu.CompilerParams(
            use_tc_tiling_on_sc=False, needs_layout_passes=False),
        scratch_types=[
            pltpu.VMEM((_TW,), jnp.int32),
            pltpu.VMEM((_TW,), jnp.int32),
            pltpu.VMEM((_TW, d), jnp.float32),
            pltpu.VMEM((_TW, d), jnp.float32),
            pltpu.VMEM((_TW * d,), jnp.float32),
            pltpu.VMEM((_TW * d,), jnp.float32),
            pltpu.SemaphoreType.DMA,
            pltpu.SemaphoreType.DMA,
            pltpu.SemaphoreType.DMA,
            pltpu.SemaphoreType.DMA,
            pltpu.SemaphoreType.DMA,
            pltpu.SemaphoreType.DMA,
        ],
    )
    def emb(table_hbm, xq_hbm, out_hbm, xb0, xb1, rw0, rw1, ob0, ob1,
            si0, si1, sg0, sg1, so0, so1):
        wid = lax.axis_index("s") * num_cores + lax.axis_index("c")
        t0 = wid * tiles_per_w

        iota = lax.iota(jnp.int32, 16)

        def start_idx(tile, xb, si):
            pltpu.async_copy(xq_hbm.at[pl.ds(tile * _TW, _TW)], xb, si)

        def wait_idx(xb, si):
            pltpu.make_async_copy(xq_hbm.at[pl.ds(0, _TW)], xb, si).wait()

        def start_gather(xb, rw, sg):
            pltpu.async_copy(table_hbm.at[xb], rw, sg)

        def wait_gather(xb, rw, sg):
            pltpu.make_async_copy(table_hbm.at[xb], rw, sg).wait()

        def wait_out(ob, so):
            for _ in range(_SUBL):
                pltpu.make_async_copy(ob.at[pl.ds(0, _TW)],
                                      out_hbm.at[pl.ds(0, _TW)], so).wait()

        def transpose_and_store(tile, rw, ob, so):
            def s_body(s, carry):
                o_base = s * _TW
                r_base = s * _LANES
                for e in range(d):
                    e_vec = jnp.full((16,), e, jnp.int32)
                    for g in range(_LANES // 16):
                        r_vec = iota + (r_base + 16 * g)
                        vals = plsc.load_gather(rw, [r_vec, e_vec])
                        ob[pl.ds(o_base + e * _LANES + 16 * g, 16)] = vals
                return carry

            lax.fori_loop(0, _SUBL, s_body, 0)

            lt = tile // bt_tiles
            bt = tile - lt * bt_tiles
            for k in range(_SUBL):
                off = (((lt * _SUBL + k) * bt_tiles) + bt) * _TW
                pltpu.async_copy(ob.at[pl.ds(k * _TW, _TW)],
                                 out_hbm.at[pl.ds(off, _TW)], so)

        start_idx(t0, xb0, si0)
        wait_idx(xb0, si0)
        start_gather(xb0, rw0, sg0)
        start_idx(t0 + 1, xb1, si1)

        def body(p, carry):
            t_a = 2 * p
            t_b = 2 * p + 1

            wait_idx(xb1, si1)
            start_gather(xb1, rw1, sg1)

            @pl.when(p >= 1)
            def _():
                wait_out(ob1, so1)

            wait_gather(xb0, rw0, sg0)

            @pl.when(t_a + 2 < tiles_per_w)
            def _():
                start_idx(t0 + t_a + 2, xb0, si0)

            @pl.when(p >= 1)
            def _():
                wait_out(ob0, so0)

            transpose_and_store(t0 + t_a, rw0, ob0, so0)

            @pl.when(t_b + 2 < tiles_per_w)
            def _():
                wait_idx(xb0, si0)

            wait_gather(xb1, rw1, sg1)

            @pl.when(t_b + 2 < tiles_per_w)
            def _():
                start_gather(xb0, rw0, sg0)

            @pl.when(t_b + 2 < tiles_per_w)
            def _():
                start_idx(t0 + t_b + 2, xb1, si1)

            transpose_and_store(t0 + t_b, rw1, ob1, so1)
            return carry

        lax.fori_loop(0, tiles_per_w // 2, body, 0)

        wait_out(ob0, so0)
        wait_out(ob1, so1)

    return emb(table, xq)


def kernel(x, table):
    b, l = x.shape
    v, d = table.shape
    assert l % _SUBL == 0 and b % _LANES == 0 and d == _SUBL
    lt_tiles = l // _SUBL
    bt_tiles = b // _LANES
    n_tiles = lt_tiles * bt_tiles

    info = plsc.get_sparse_core_info()
    num_workers = info.num_cores * info.num_subcores
    tiles_per_w = n_tiles // num_workers
    assert n_tiles % num_workers == 0 and tiles_per_w % 2 == 0

    xq = (x.T.reshape(lt_tiles, _SUBL, bt_tiles, _LANES)
          .transpose(0, 2, 1, 3).reshape(b * l))

    out_flat = _emb_lookup(table, xq, n_tiles, tiles_per_w, bt_tiles,
                           info.num_cores)

    out = (out_flat.reshape(lt_tiles * _SUBL, bt_tiles, _SUBL, _LANES)
           .transpose(1, 3, 0, 2).reshape(b, l * d))
    return out


# speedup vs baseline: 126.8414x; 1.6502x over previous
import functools

import jax
import jax.numpy as jnp
from jax import lax
from jax.experimental import pallas as pl
from jax.experimental.pallas import tpu as pltpu
from jax.experimental.pallas import tpu_sc as plsc

_LANES = 128
_SUBL = 8
_TW = _LANES * _SUBL


def _table_rowmajor(table_t, num_cores):
    d, v = table_t.shape
    n_full = v // _LANES
    v_tail = v - n_full * _LANES
    num_workers = num_cores * 16
    steps = pl.cdiv(n_full, num_workers)
    n_pairs = (steps + 1) // 2
    mesh = plsc.VectorSubcoreMesh(core_axis_name="c", subcore_axis_name="s")

    @functools.partial(
        pl.kernel,
        mesh=mesh,
        out_type=jax.ShapeDtypeStruct((v * d,), jnp.float32),
        compiler_params=pltpu.CompilerParams(needs_layout_passes=False),
        scratch_types=[
            pltpu.VMEM((d, _LANES), jnp.float32),
            pltpu.VMEM((d, _LANES), jnp.float32),
            pltpu.VMEM((_TW,), jnp.float32),
            pltpu.VMEM((_TW,), jnp.float32),
            pltpu.SemaphoreType.DMA,
            pltpu.SemaphoreType.DMA,
            pltpu.SemaphoreType.DMA,
            pltpu.SemaphoreType.DMA,
        ],
    )
    def tr(tbl_hbm, out_hbm, tb0, tb1, ob0, ob1, si0, si1, so0, so1):
        wid = lax.axis_index("s") * num_cores + lax.axis_index("c")

        iota = lax.iota(jnp.int32, 16)
        e_vec = lax.rem(iota, d)
        q_vec = lax.div(iota, d)

        def start_in(vt, tb, si, n=_LANES):
            pltpu.async_copy(tbl_hbm.at[:, pl.ds(vt * _LANES, n)],
                             tb.at[:, pl.ds(0, n)], si)

        def wait_in(tb, si, n=_LANES):
            pltpu.make_async_copy(tbl_hbm.at[:, pl.ds(0, n)],
                                  tb.at[:, pl.ds(0, n)], si).wait()

        def wait_out(ob, so, n=_LANES):
            pltpu.make_async_copy(ob.at[pl.ds(0, n * d)],
                                  out_hbm.at[pl.ds(0, n * d)], so).wait()

        def transpose_store(vt, tb, ob, so, n=_LANES):
            def q_body(q, carry):
                j_vec = q_vec + 2 * q
                vals = plsc.load_gather(tb, [e_vec, j_vec])
                ob[pl.ds(16 * q, 16)] = vals
                return carry

            lax.fori_loop(0, (n * d) // 16, q_body, 0, unroll=8)
            pltpu.async_copy(ob.at[pl.ds(0, n * d)],
                             out_hbm.at[pl.ds(vt * _TW, n * d)], so)

        def slot(i, vt, tb, ob, si, so):
            @pl.when(vt < n_full)
            def _():
                wait_in(tb, si)

                @pl.when(i >= 2)
                def _():
                    wait_out(ob, so)

                transpose_store(vt, tb, ob, so)
                vt_next = vt + 2 * num_workers

                @pl.when(vt_next < n_full)
                def _():
                    start_in(vt_next, tb, si)

        @pl.when(wid < n_full)
        def _():
            start_in(wid, tb0, si0)

        @pl.when(wid + num_workers < n_full)
        def _():
            start_in(wid + num_workers, tb1, si1)

        def body(p, carry):
            slot(2 * p, wid + 2 * p * num_workers, tb0, ob0, si0, so0)
            slot(2 * p + 1, wid + (2 * p + 1) * num_workers, tb1, ob1,
                 si1, so1)
            return carry

        lax.fori_loop(0, n_pairs, body, 0)

        @pl.when(wid < n_full)
        def _():
            wait_out(ob0, so0)

        @pl.when(wid + num_workers < n_full)
        def _():
            wait_out(ob1, so1)

        if v_tail:
            @pl.when(wid == 0)
            def _():
                for e in range(d):
                    pltpu.sync_copy(
                        tbl_hbm.at[e, pl.ds(n_full * _LANES, v_tail)],
                        tb0.at[e, pl.ds(0, v_tail)])

                def q_body(q, carry):
                    j_vec = q_vec + 2 * q
                    vals = plsc.load_gather(tb0, [e_vec, j_vec])
                    ob0[pl.ds(16 * q, 16)] = vals
                    return carry

                lax.fori_loop(0, (v_tail * d) // 16, q_body, 0, unroll=8)
                pltpu.sync_copy(ob0.at[pl.ds(0, v_tail * d)],
                                out_hbm.at[pl.ds(n_full * _TW, v_tail * d)])

    return tr(table_t)


def _emb_lookup(table, xq, n_tiles, tiles_per_w, bt_tiles, num_cores):
    v, d = table.shape
    n_out = xq.shape[0] * d
    mesh = plsc.VectorSubcoreMesh(core_axis_name="c", subcore_axis_name="s")

    @functools.partial(
        pl.kernel,
        mesh=mesh,
        out_type=jax.ShapeDtypeStruct((n_out,), jnp.float32),
        compiler_params=pltpu.CompilerParams(
            use_tc_tiling_on_sc=False, needs_layout_passes=False),
        scratch_types=[
            pltpu.VMEM((_TW,), jnp.int32),
            pltpu.VMEM((_TW,), jnp.int32),
            pltpu.VMEM((_TW, d), jnp.float32),
            pltpu.VMEM((_TW, d), jnp.float32),
            pltpu.VMEM((_TW * d,), jnp.float32),
            pltpu.VMEM((_TW * d,), jnp.float32),
            pltpu.SemaphoreType.DMA,
            pltpu.SemaphoreType.DMA,
            pltpu.SemaphoreType.DMA,
            pltpu.SemaphoreType.DMA,
            pltpu.SemaphoreType.DMA,
            pltpu.SemaphoreType.DMA,
        ],
    )
    def emb(table_hbm, xq_hbm, out_hbm, xb0, xb1, rw0, rw1, ob0, ob1,
            si0, si1, sg0, sg1, so0, so1):
        wid = lax.axis_index("s") * num_cores + lax.axis_index("c")
        t0 = wid * tiles_per_w

        iota = lax.iota(jnp.int32, 16)

        def start_idx(tile, xb, si):
            pltpu.async_copy(xq_hbm.at[pl.ds(tile * _TW, _TW)], xb, si)

        def wait_idx(xb, si):
            pltpu.make_async_copy(xq_hbm.at[pl.ds(0, _TW)], xb, si).wait()

        def start_gather(xb, rw, sg):
            pltpu.async_copy(table_hbm.at[xb], rw, sg)

        def wait_gather(xb, rw, sg):
            pltpu.make_async_copy(table_hbm.at[xb], rw, sg).wait()

        def wait_out(ob, so):
            for _ in range(_SUBL):
                pltpu.make_async_copy(ob.at[pl.ds(0, _TW)],
                                      out_hbm.at[pl.ds(0, _TW)], so).wait()

        def transpose_and_store(tile, rw, ob, so):
            def s_body(s, carry):
                o_base = s * _TW
                r_base = s * _LANES
                for e in range(d):
                    e_vec = jnp.full((16,), e, jnp.int32)
                    for g in range(_LANES // 16):
                        r_vec = iota + (r_base + 16 * g)
                        vals = plsc.load_gather(rw, [r_vec, e_vec])
                        ob[pl.ds(o_base + e * _LANES + 16 * g, 16)] = vals
                return carry

            lax.fori_loop(0, _SUBL, s_body, 0)

            lt = tile // bt_tiles
            bt = tile - lt * bt_tiles
            for k in range(_SUBL):
                off = (((lt * _SUBL + k) * bt_tiles) + bt) * _TW
                pltpu.async_copy(ob.at[pl.ds(k * _TW, _TW)],
                                 out_hbm.at[pl.ds(off, _TW)], so)

        start_idx(t0, xb0, si0)
        wait_idx(xb0, si0)
        start_gather(xb0, rw0, sg0)
        start_idx(t0 + 1, xb1, si1)

        def body(p, carry):
            t_a = 2 * p
            t_b = 2 * p + 1

            wait_idx(xb1, si1)
            start_gather(xb1, rw1, sg1)

            @pl.when(p >= 1)
            def _():
                wait_out(ob1, so1)

            wait_gather(xb0, rw0, sg0)

            @pl.when(t_a + 2 < tiles_per_w)
            def _():
                start_idx(t0 + t_a + 2, xb0, si0)

            @pl.when(p >= 1)
            def _():
                wait_out(ob0, so0)

            transpose_and_store(t0 + t_a, rw0, ob0, so0)

            @pl.when(t_b + 2 < tiles_per_w)
            def _():
                wait_idx(xb0, si0)

            wait_gather(xb1, rw1, sg1)

            @pl.when(t_b + 2 < tiles_per_w)
            def _():
                start_gather(xb0, rw0, sg0)

            @pl.when(t_b + 2 < tiles_per_w)
            def _():
                start_idx(t0 + t_b + 2, xb1, si1)

            transpose_and_store(t0 + t_b, rw1, ob1, so1)
            return carry

        lax.fori_loop(0, tiles_per_w // 2, body, 0)

        wait_out(ob0, so0)
        wait_out(ob1, so1)

    return emb(table, xq)


def kernel(x, table):
    b, l = x.shape
    v, d = table.shape
    assert l % _SUBL == 0 and b % _LANES == 0 and d == _SUBL
    lt_tiles = l // _SUBL
    bt_tiles = b // _LANES
    n_tiles = lt_tiles * bt_tiles

    info = plsc.get_sparse_core_info()
    num_workers = info.num_cores * info.num_subcores
    tiles_per_w = n_tiles // num_workers
    assert n_tiles % num_workers == 0 and tiles_per_w % 2 == 0

    xq = (x.T.reshape(lt_tiles, _SUBL, bt_tiles, _LANES)
          .transpose(0, 2, 1, 3).reshape(b * l))

    tbl_flat = _table_rowmajor(table.T, info.num_cores)
    tbl_rm = tbl_flat.reshape(tbl_flat.shape[0] // d, d)

    out_flat = _emb_lookup(tbl_rm, xq, n_tiles, tiles_per_w, bt_tiles,
                           info.num_cores)

    out = (out_flat.reshape(lt_tiles * _SUBL, bt_tiles, _SUBL, _LANES)
           .transpose(1, 3, 0, 2).reshape(b, l * d))
    return out


# speedup vs baseline: 127.4447x; 1.0048x over previous
import functools

import jax
import jax.numpy as jnp
from jax import lax
from jax.experimental import pallas as pl
from jax.experimental.pallas import tpu as pltpu
from jax.experimental.pallas import tpu_sc as plsc

_LANES = 128
_SUBL = 8
_TW = _LANES * _SUBL


def _table_rowmajor(table_t, num_cores):
    d, v = table_t.shape
    n_full = v // _LANES
    v_tail = v - n_full * _LANES
    num_workers = num_cores * 16
    steps = pl.cdiv(n_full, num_workers)
    n_pairs = (steps + 1) // 2
    mesh = plsc.VectorSubcoreMesh(core_axis_name="c", subcore_axis_name="s")

    @functools.partial(
        pl.kernel,
        mesh=mesh,
        out_type=jax.ShapeDtypeStruct((v * d,), jnp.float32),
        compiler_params=pltpu.CompilerParams(needs_layout_passes=False),
        scratch_types=[
            pltpu.VMEM((d, _LANES), jnp.float32),
            pltpu.VMEM((d, _LANES), jnp.float32),
            pltpu.VMEM((_TW,), jnp.float32),
            pltpu.VMEM((_TW,), jnp.float32),
            pltpu.SemaphoreType.DMA,
            pltpu.SemaphoreType.DMA,
            pltpu.SemaphoreType.DMA,
            pltpu.SemaphoreType.DMA,
        ],
    )
    def tr(tbl_hbm, out_hbm, tb0, tb1, ob0, ob1, si0, si1, so0, so1):
        wid = lax.axis_index("s") * num_cores + lax.axis_index("c")

        iota = lax.iota(jnp.int32, 16)
        e_vec = lax.rem(iota, d)
        q_vec = lax.div(iota, d)

        def start_in(vt, tb, si, n=_LANES):
            pltpu.async_copy(tbl_hbm.at[:, pl.ds(vt * _LANES, n)],
                             tb.at[:, pl.ds(0, n)], si)

        def wait_in(tb, si, n=_LANES):
            pltpu.make_async_copy(tbl_hbm.at[:, pl.ds(0, n)],
                                  tb.at[:, pl.ds(0, n)], si).wait()

        def wait_out(ob, so, n=_LANES):
            pltpu.make_async_copy(ob.at[pl.ds(0, n * d)],
                                  out_hbm.at[pl.ds(0, n * d)], so).wait()

        def transpose_store(vt, tb, ob, so, n=_LANES):
            def q_body(q, carry):
                j_vec = q_vec + 2 * q
                vals = plsc.load_gather(tb, [e_vec, j_vec])
                ob[pl.ds(16 * q, 16)] = vals
                return carry

            lax.fori_loop(0, (n * d) // 16, q_body, 0, unroll=8)
            pltpu.async_copy(ob.at[pl.ds(0, n * d)],
                             out_hbm.at[pl.ds(vt * _TW, n * d)], so)

        def slot(i, vt, tb, ob, si, so):
            @pl.when(vt < n_full)
            def _():
                wait_in(tb, si)

                @pl.when(i >= 2)
                def _():
                    wait_out(ob, so)

                transpose_store(vt, tb, ob, so)
                vt_next = vt + 2 * num_workers

                @pl.when(vt_next < n_full)
                def _():
                    start_in(vt_next, tb, si)

        @pl.when(wid < n_full)
        def _():
            start_in(wid, tb0, si0)

        @pl.when(wid + num_workers < n_full)
        def _():
            start_in(wid + num_workers, tb1, si1)

        def body(p, carry):
            slot(2 * p, wid + 2 * p * num_workers, tb0, ob0, si0, so0)
            slot(2 * p + 1, wid + (2 * p + 1) * num_workers, tb1, ob1,
                 si1, so1)
            return carry

        lax.fori_loop(0, n_pairs, body, 0)

        @pl.when(wid < n_full)
        def _():
            wait_out(ob0, so0)

        @pl.when(wid + num_workers < n_full)
        def _():
            wait_out(ob1, so1)

        if v_tail:
            @pl.when(wid == 0)
            def _():
                for e in range(d):
                    pltpu.sync_copy(
                        tbl_hbm.at[e, pl.ds(n_full * _LANES, v_tail)],
                        tb0.at[e, pl.ds(0, v_tail)])

                def q_body(q, carry):
                    j_vec = q_vec + 2 * q
                    vals = plsc.load_gather(tb0, [e_vec, j_vec])
                    ob0[pl.ds(16 * q, 16)] = vals
                    return carry

                lax.fori_loop(0, (v_tail * d) // 16, q_body, 0, unroll=8)
                pltpu.sync_copy(ob0.at[pl.ds(0, v_tail * d)],
                                out_hbm.at[pl.ds(n_full * _TW, v_tail * d)])

    return tr(table_t)


def _emb_lookup(table, xq, tiles_per_w, bt_tiles, num_cores):
    v, d = table.shape
    n_out = xq.shape[0] * d
    pairs_per_w = tiles_per_w // 2
    pw = 2 * _TW
    mesh = plsc.VectorSubcoreMesh(core_axis_name="c", subcore_axis_name="s")

    @functools.partial(
        pl.kernel,
        mesh=mesh,
        out_type=jax.ShapeDtypeStruct((n_out,), jnp.float32),
        compiler_params=pltpu.CompilerParams(
            use_tc_tiling_on_sc=False, needs_layout_passes=False),
        scratch_types=[
            pltpu.VMEM((pw,), jnp.int32),
            pltpu.VMEM((pw,), jnp.int32),
            pltpu.VMEM((pw, d), jnp.float32),
            pltpu.VMEM((pw, d), jnp.float32),
            pltpu.VMEM((pw * d,), jnp.float32),
            pltpu.VMEM((pw * d,), jnp.float32),
            pltpu.SemaphoreType.DMA,
            pltpu.SemaphoreType.DMA,
            pltpu.SemaphoreType.DMA,
            pltpu.SemaphoreType.DMA,
            pltpu.SemaphoreType.DMA,
            pltpu.SemaphoreType.DMA,
        ],
    )
    def emb(table_hbm, xq_hbm, out_hbm, xb0, xb1, rw0, rw1, ob0, ob1,
            si0, si1, sg0, sg1, so0, so1):
        wid = lax.axis_index("s") * num_cores + lax.axis_index("c")
        p0 = wid * pairs_per_w

        iota = lax.iota(jnp.int32, 16)
        e_vecs = [jnp.full((16,), e, jnp.int32) for e in range(d)]

        def start_idx(pair, xb, si):
            pltpu.async_copy(xq_hbm.at[pl.ds(pair * pw, pw)], xb, si)

        def wait_idx(xb, si):
            pltpu.make_async_copy(xq_hbm.at[pl.ds(0, pw)], xb, si).wait()

        def start_gather(xb, rw, sg):
            pltpu.async_copy(table_hbm.at[xb], rw, sg)

        def wait_gather(xb, rw, sg):
            pltpu.make_async_copy(table_hbm.at[xb], rw, sg).wait()

        def wait_out(ob, so):
            for _ in range(_SUBL):
                pltpu.make_async_copy(ob.at[pl.ds(0, 2 * _TW)],
                                      out_hbm.at[pl.ds(0, 2 * _TW)],
                                      so).wait()

        def transpose_and_store(pair, rw, ob, so):
            def q_body(q, carry):
                r_base = q * _LANES
                o_base = (lax.rem(q, _SUBL) * 2 * _TW
                          + lax.div(q, _SUBL) * _TW)
                for g in range(_LANES // 16):
                    rws = rw.at[pl.ds(r_base + 16 * g, 16), :]
                    for e in range(d):
                        vals = plsc.load_gather(rws, [iota, e_vecs[e]])
                        ob[pl.ds(o_base + e * _LANES + 16 * g, 16)] = vals
                return carry

            lax.fori_loop(0, 2 * _SUBL, q_body, 0)

            tile = 2 * pair
            lt = tile // bt_tiles
            bt = tile - lt * bt_tiles
            for k in range(_SUBL):
                off = (((lt * _SUBL + k) * bt_tiles) + bt) * _TW
                pltpu.async_copy(ob.at[pl.ds(k * 2 * _TW, 2 * _TW)],
                                 out_hbm.at[pl.ds(off, 2 * _TW)], so)

        start_idx(p0, xb0, si0)
        wait_idx(xb0, si0)
        start_gather(xb0, rw0, sg0)
        start_idx(p0 + 1, xb1, si1)

        def body(p, carry):
            t_a = 2 * p
            t_b = 2 * p + 1

            wait_idx(xb1, si1)
            start_gather(xb1, rw1, sg1)

            @pl.when(p >= 1)
            def _():
                wait_out(ob1, so1)

            wait_gather(xb0, rw0, sg0)

            @pl.when(t_a + 2 < pairs_per_w)
            def _():
                start_idx(p0 + t_a + 2, xb0, si0)

            @pl.when(p >= 1)
            def _():
                wait_out(ob0, so0)

            transpose_and_store(p0 + t_a, rw0, ob0, so0)

            @pl.when(t_b + 2 < pairs_per_w)
            def _():
                wait_idx(xb0, si0)

            wait_gather(xb1, rw1, sg1)

            @pl.when(t_b + 2 < pairs_per_w)
            def _():
                start_gather(xb0, rw0, sg0)

            @pl.when(t_b + 2 < pairs_per_w)
            def _():
                start_idx(p0 + t_b + 2, xb1, si1)

            transpose_and_store(p0 + t_b, rw1, ob1, so1)
            return carry

        lax.fori_loop(0, pairs_per_w // 2, body, 0)

        wait_out(ob0, so0)
        wait_out(ob1, so1)

    return emb(table, xq)


def kernel(x, table):
    b, l = x.shape
    v, d = table.shape
    assert l % _SUBL == 0 and b % _LANES == 0 and d == _SUBL
    lt_tiles = l // _SUBL
    bt_tiles = b // _LANES
    n_tiles = lt_tiles * bt_tiles

    info = plsc.get_sparse_core_info()
    num_workers = info.num_cores * info.num_subcores
    tiles_per_w = n_tiles // num_workers
    assert n_tiles % num_workers == 0 and tiles_per_w % 2 == 0

    xq = (x.T.reshape(lt_tiles, _SUBL, bt_tiles, _LANES)
          .transpose(0, 2, 1, 3).reshape(b * l))

    tbl_flat = _table_rowmajor(table.T, info.num_cores)
    tbl_rm = tbl_flat.reshape(tbl_flat.shape[0] // d, d)

    out_flat = _emb_lookup(tbl_rm, xq, tiles_per_w, bt_tiles,
                           info.num_cores)

    out = (out_flat.reshape(lt_tiles * _SUBL, bt_tiles, _SUBL, _LANES)
           .transpose(1, 3, 0, 2).reshape(b, l * d))
    return out


# speedup vs baseline: 132.7327x; 1.0415x over previous
import functools

import jax
import jax.numpy as jnp
from jax import lax
from jax.experimental import pallas as pl
from jax.experimental.pallas import tpu as pltpu
from jax.experimental.pallas import tpu_sc as plsc

_LANES = 128
_SUBL = 8
_TW = _LANES * _SUBL


def _table_rowmajor(table_t, num_cores):
    d, v = table_t.shape
    n_full = v // _LANES
    v_tail = v - n_full * _LANES
    bt_sz = 4
    bw = bt_sz * _LANES
    n_batch = n_full // bt_sz
    assert n_full % bt_sz == 0
    num_workers = num_cores * 16
    n_pairs = (pl.cdiv(n_batch, num_workers) + 1) // 2
    mesh = plsc.VectorSubcoreMesh(core_axis_name="c", subcore_axis_name="s")

    @functools.partial(
        pl.kernel,
        mesh=mesh,
        out_type=jax.ShapeDtypeStruct((v * d,), jnp.float32),
        compiler_params=pltpu.CompilerParams(needs_layout_passes=False),
        scratch_types=[
            pltpu.VMEM((d, bw), jnp.float32),
            pltpu.VMEM((d, bw), jnp.float32),
            pltpu.VMEM((bw * d,), jnp.float32),
            pltpu.VMEM((bw * d,), jnp.float32),
            pltpu.SemaphoreType.DMA,
            pltpu.SemaphoreType.DMA,
            pltpu.SemaphoreType.DMA,
            pltpu.SemaphoreType.DMA,
        ],
    )
    def tr(tbl_hbm, out_hbm, tb0, tb1, ob0, ob1, si0, si1, so0, so1):
        wid = lax.axis_index("s") * num_cores + lax.axis_index("c")

        iota = lax.iota(jnp.int32, 16)
        e_vec = lax.rem(iota, d)
        q_vec = lax.div(iota, d)

        def start_in(b, tb, si):
            pltpu.async_copy(tbl_hbm.at[:, pl.ds(b * bw, bw)], tb, si)

        def wait_in(tb, si):
            pltpu.make_async_copy(tbl_hbm.at[:, pl.ds(0, bw)], tb,
                                  si).wait()

        def wait_out(ob, so):
            pltpu.make_async_copy(ob, out_hbm.at[pl.ds(0, bw * d)],
                                  so).wait()

        def transpose_store(b, tb, ob, so):
            def q_body(q, carry):
                j_vec = q_vec + 2 * q
                vals = plsc.load_gather(tb, [e_vec, j_vec])
                ob[pl.ds(16 * q, 16)] = vals
                return carry

            lax.fori_loop(0, (bw * d) // 16, q_body, 0, unroll=8)
            pltpu.async_copy(ob, out_hbm.at[pl.ds(b * bw * d, bw * d)], so)

        def slot(i, b, tb, ob, si, so):
            @pl.when(b < n_batch)
            def _():
                wait_in(tb, si)

                @pl.when(i >= 2)
                def _():
                    wait_out(ob, so)

                transpose_store(b, tb, ob, so)
                b_next = b + 2 * num_workers

                @pl.when(b_next < n_batch)
                def _():
                    start_in(b_next, tb, si)

        @pl.when(wid < n_batch)
        def _():
            start_in(wid, tb0, si0)

        @pl.when(wid + num_workers < n_batch)
        def _():
            start_in(wid + num_workers, tb1, si1)

        def body(p, carry):
            slot(2 * p, wid + 2 * p * num_workers, tb0, ob0, si0, so0)
            slot(2 * p + 1, wid + (2 * p + 1) * num_workers, tb1, ob1,
                 si1, so1)
            return carry

        lax.fori_loop(0, n_pairs, body, 0)

        @pl.when(wid < n_batch)
        def _():
            wait_out(ob0, so0)

        @pl.when(wid + num_workers < n_batch)
        def _():
            wait_out(ob1, so1)

        if v_tail:
            @pl.when(wid == 0)
            def _():
                for e in range(d):
                    pltpu.sync_copy(
                        tbl_hbm.at[e, pl.ds(n_full * _LANES, v_tail)],
                        tb0.at[e, pl.ds(0, v_tail)])

                def q_body(q, carry):
                    j_vec = q_vec + 2 * q
                    vals = plsc.load_gather(tb0, [e_vec, j_vec])
                    ob0[pl.ds(16 * q, 16)] = vals
                    return carry

                lax.fori_loop(0, (v_tail * d) // 16, q_body, 0, unroll=8)
                pltpu.sync_copy(ob0.at[pl.ds(0, v_tail * d)],
                                out_hbm.at[pl.ds(n_full * _TW, v_tail * d)])

    return tr(table_t)


def _emb_lookup(table, xq, tiles_per_w, bt_tiles, num_cores):
    v, d = table.shape
    n_out = xq.shape[0] * d
    pairs_per_w = tiles_per_w // 2
    pw = 2 * _TW
    mesh = plsc.VectorSubcoreMesh(core_axis_name="c", subcore_axis_name="s")

    @functools.partial(
        pl.kernel,
        mesh=mesh,
        out_type=jax.ShapeDtypeStruct((n_out,), jnp.float32),
        compiler_params=pltpu.CompilerParams(
            use_tc_tiling_on_sc=False, needs_layout_passes=False),
        scratch_types=[
            pltpu.VMEM((pw,), jnp.int32),
            pltpu.VMEM((pw,), jnp.int32),
            pltpu.VMEM((pw, d), jnp.float32),
            pltpu.VMEM((pw, d), jnp.float32),
            pltpu.VMEM((pw * d,), jnp.float32),
            pltpu.VMEM((pw * d,), jnp.float32),
            pltpu.SemaphoreType.DMA,
            pltpu.SemaphoreType.DMA,
            pltpu.SemaphoreType.DMA,
            pltpu.SemaphoreType.DMA,
            pltpu.SemaphoreType.DMA,
            pltpu.SemaphoreType.DMA,
        ],
    )
    def emb(table_hbm, xq_hbm, out_hbm, xb0, xb1, rw0, rw1, ob0, ob1,
            si0, si1, sg0, sg1, so0, so1):
        wid = lax.axis_index("s") * num_cores + lax.axis_index("c")
        p0 = wid * pairs_per_w

        iota = lax.iota(jnp.int32, 16)
        e_vecs = [jnp.full((16,), e, jnp.int32) for e in range(d)]

        def start_idx(pair, xb, si):
            pltpu.async_copy(xq_hbm.at[pl.ds(pair * pw, pw)], xb, si)

        def wait_idx(xb, si):
            pltpu.make_async_copy(xq_hbm.at[pl.ds(0, pw)], xb, si).wait()

        def start_gather(xb, rw, sg):
            pltpu.async_copy(table_hbm.at[xb], rw, sg)

        def wait_gather(xb, rw, sg):
            pltpu.make_async_copy(table_hbm.at[xb], rw, sg).wait()

        def wait_out(ob, so):
            for _ in range(_SUBL):
                pltpu.make_async_copy(ob.at[pl.ds(0, 2 * _TW)],
                                      out_hbm.at[pl.ds(0, 2 * _TW)],
                                      so).wait()

        def transpose_and_store(pair, rw, ob, so):
            def q_body(q, carry):
                r_base = q * _LANES
                o_base = (lax.rem(q, _SUBL) * 2 * _TW
                          + lax.div(q, _SUBL) * _TW)
                for g in range(_LANES // 16):
                    rws = rw.at[pl.ds(r_base + 16 * g, 16), :]
                    for e in range(d):
                        vals = plsc.load_gather(rws, [iota, e_vecs[e]])
                        ob[pl.ds(o_base + e * _LANES + 16 * g, 16)] = vals
                return carry

            lax.fori_loop(0, 2 * _SUBL, q_body, 0)

            tile = 2 * pair
            lt = tile // bt_tiles
            bt = tile - lt * bt_tiles
            for k in range(_SUBL):
                off = (((lt * _SUBL + k) * bt_tiles) + bt) * _TW
                pltpu.async_copy(ob.at[pl.ds(k * 2 * _TW, 2 * _TW)],
                                 out_hbm.at[pl.ds(off, 2 * _TW)], so)

        start_idx(p0, xb0, si0)
        wait_idx(xb0, si0)
        start_gather(xb0, rw0, sg0)
        start_idx(p0 + 1, xb1, si1)

        def body(p, carry):
            t_a = 2 * p
            t_b = 2 * p + 1

            wait_idx(xb1, si1)
            start_gather(xb1, rw1, sg1)

            @pl.when(p >= 1)
            def _():
                wait_out(ob1, so1)

            wait_gather(xb0, rw0, sg0)

            @pl.when(t_a + 2 < pairs_per_w)
            def _():
                start_idx(p0 + t_a + 2, xb0, si0)

            @pl.when(p >= 1)
            def _():
                wait_out(ob0, so0)

            transpose_and_store(p0 + t_a, rw0, ob0, so0)

            @pl.when(t_b + 2 < pairs_per_w)
            def _():
                wait_idx(xb0, si0)

            wait_gather(xb1, rw1, sg1)

            @pl.when(t_b + 2 < pairs_per_w)
            def _():
                start_gather(xb0, rw0, sg0)

            @pl.when(t_b + 2 < pairs_per_w)
            def _():
                start_idx(p0 + t_b + 2, xb1, si1)

            transpose_and_store(p0 + t_b, rw1, ob1, so1)
            return carry

        lax.fori_loop(0, pairs_per_w // 2, body, 0)

        wait_out(ob0, so0)
        wait_out(ob1, so1)

    return emb(table, xq)


def kernel(x, table):
    b, l = x.shape
    v, d = table.shape
    assert l % _SUBL == 0 and b % _LANES == 0 and d == _SUBL
    lt_tiles = l // _SUBL
    bt_tiles = b // _LANES
    n_tiles = lt_tiles * bt_tiles

    info = plsc.get_sparse_core_info()
    num_workers = info.num_cores * info.num_subcores
    tiles_per_w = n_tiles // num_workers
    assert n_tiles % num_workers == 0 and tiles_per_w % 2 == 0

    xq = (x.T.reshape(lt_tiles, _SUBL, bt_tiles, _LANES)
          .transpose(0, 2, 1, 3).reshape(b * l))

    tbl_flat = _table_rowmajor(table.T, info.num_cores)
    tbl_rm = tbl_flat.reshape(tbl_flat.shape[0] // d, d)

    out_flat = _emb_lookup(tbl_rm, xq, tiles_per_w, bt_tiles,
                           info.num_cores)

    out = (out_flat.reshape(lt_tiles * _SUBL, bt_tiles, _SUBL, _LANES)
           .transpose(1, 3, 0, 2).reshape(b, l * d))
    return out


# speedup vs baseline: 132.7370x; 1.0000x over previous
import functools

import jax
import jax.numpy as jnp
from jax import lax
from jax.experimental import pallas as pl
from jax.experimental.pallas import tpu as pltpu
from jax.experimental.pallas import tpu_sc as plsc

_LANES = 128
_SUBL = 8
_TW = _LANES * _SUBL


def _table_rowmajor(table_t, num_cores):
    d, v = table_t.shape
    n_full = v // _LANES
    v_tail = v - n_full * _LANES
    bt_sz = 4
    bw = bt_sz * _LANES
    n_batch = n_full // bt_sz
    assert n_full % bt_sz == 0
    num_workers = num_cores * 16
    n_pairs = (pl.cdiv(n_batch, num_workers) + 1) // 2
    mesh = plsc.VectorSubcoreMesh(core_axis_name="c", subcore_axis_name="s")

    @functools.partial(
        pl.kernel,
        mesh=mesh,
        out_type=jax.ShapeDtypeStruct((v * d,), jnp.float32),
        compiler_params=pltpu.CompilerParams(needs_layout_passes=False),
        scratch_types=[
            pltpu.VMEM((d, bw), jnp.float32),
            pltpu.VMEM((d, bw), jnp.float32),
            pltpu.VMEM((bw * d,), jnp.float32),
            pltpu.VMEM((bw * d,), jnp.float32),
            pltpu.SemaphoreType.DMA,
            pltpu.SemaphoreType.DMA,
            pltpu.SemaphoreType.DMA,
            pltpu.SemaphoreType.DMA,
        ],
    )
    def tr(tbl_hbm, out_hbm, tb0, tb1, ob0, ob1, si0, si1, so0, so1):
        wid = lax.axis_index("s") * num_cores + lax.axis_index("c")

        iota = lax.iota(jnp.int32, 16)
        e_vec = lax.rem(iota, d)
        q_vec = lax.div(iota, d)

        def start_in(b, tb, si):
            pltpu.async_copy(tbl_hbm.at[:, pl.ds(b * bw, bw)], tb, si)

        def wait_in(tb, si):
            pltpu.make_async_copy(tbl_hbm.at[:, pl.ds(0, bw)], tb,
                                  si).wait()

        def wait_out(ob, so):
            pltpu.make_async_copy(ob, out_hbm.at[pl.ds(0, bw * d)],
                                  so).wait()

        def transpose_store(b, tb, ob, so):
            def q_body(q, carry):
                j_vec = q_vec + 2 * q
                vals = plsc.load_gather(tb, [e_vec, j_vec])
                ob[pl.ds(16 * q, 16)] = vals
                return carry

            lax.fori_loop(0, (bw * d) // 16, q_body, 0, unroll=8)
            pltpu.async_copy(ob, out_hbm.at[pl.ds(b * bw * d, bw * d)], so)

        def slot(i, b, tb, ob, si, so):
            @pl.when(b < n_batch)
            def _():
                wait_in(tb, si)

                @pl.when(i >= 2)
                def _():
                    wait_out(ob, so)

                transpose_store(b, tb, ob, so)
                b_next = b + 2 * num_workers

                @pl.when(b_next < n_batch)
                def _():
                    start_in(b_next, tb, si)

        @pl.when(wid < n_batch)
        def _():
            start_in(wid, tb0, si0)

        @pl.when(wid + num_workers < n_batch)
        def _():
            start_in(wid + num_workers, tb1, si1)

        def body(p, carry):
            slot(2 * p, wid + 2 * p * num_workers, tb0, ob0, si0, so0)
            slot(2 * p + 1, wid + (2 * p + 1) * num_workers, tb1, ob1,
                 si1, so1)
            return carry

        lax.fori_loop(0, n_pairs, body, 0)

        @pl.when(wid < n_batch)
        def _():
            wait_out(ob0, so0)

        @pl.when(wid + num_workers < n_batch)
        def _():
            wait_out(ob1, so1)

        if v_tail:
            @pl.when(wid == 0)
            def _():
                for e in range(d):
                    pltpu.sync_copy(
                        tbl_hbm.at[e, pl.ds(n_full * _LANES, v_tail)],
                        tb0.at[e, pl.ds(0, v_tail)])

                def q_body(q, carry):
                    j_vec = q_vec + 2 * q
                    vals = plsc.load_gather(tb0, [e_vec, j_vec])
                    ob0[pl.ds(16 * q, 16)] = vals
                    return carry

                lax.fori_loop(0, (v_tail * d) // 16, q_body, 0, unroll=8)
                pltpu.sync_copy(ob0.at[pl.ds(0, v_tail * d)],
                                out_hbm.at[pl.ds(n_full * _TW, v_tail * d)])

    return tr(table_t)


def _emb_lookup(table, xq, tiles_per_w, bt_tiles, num_cores):
    v, d = table.shape
    n_out = xq.shape[0] * d
    pairs_per_w = tiles_per_w // 2
    pw = 2 * _TW
    mesh = plsc.VectorSubcoreMesh(core_axis_name="c", subcore_axis_name="s")

    @functools.partial(
        pl.kernel,
        mesh=mesh,
        out_type=jax.ShapeDtypeStruct((n_out,), jnp.float32),
        compiler_params=pltpu.CompilerParams(
            use_tc_tiling_on_sc=False, needs_layout_passes=False),
        scratch_types=[
            pltpu.VMEM((pw,), jnp.int32),
            pltpu.VMEM((pw,), jnp.int32),
            pltpu.VMEM((pw, d), jnp.float32),
            pltpu.VMEM((pw, d), jnp.float32),
            pltpu.VMEM((pw * d,), jnp.float32),
            pltpu.VMEM((pw * d,), jnp.float32),
            pltpu.SemaphoreType.DMA,
            pltpu.SemaphoreType.DMA,
            pltpu.SemaphoreType.DMA,
            pltpu.SemaphoreType.DMA,
            pltpu.SemaphoreType.DMA,
            pltpu.SemaphoreType.DMA,
        ],
    )
    def emb(table_hbm, xq_hbm, out_hbm, xb0, xb1, rw0, rw1, ob0, ob1,
            si0, si1, sg0, sg1, so0, so1):
        wid = lax.axis_index("s") * num_cores + lax.axis_index("c")
        p0 = wid * pairs_per_w

        iota = lax.iota(jnp.int32, 16)
        e_vecs = [jnp.full((16,), e, jnp.int32) for e in range(d)]

        def start_idx(pair, xb, si):
            pltpu.async_copy(xq_hbm.at[pl.ds(pair * pw, pw)], xb, si)

        def wait_idx(xb, si):
            pltpu.make_async_copy(xq_hbm.at[pl.ds(0, pw)], xb, si).wait()

        def start_gather(xb, rw, sg):
            pltpu.async_copy(table_hbm.at[xb], rw, sg)

        def wait_gather(xb, rw, sg):
            pltpu.make_async_copy(table_hbm.at[xb], rw, sg).wait()

        def wait_out(ob, so):
            for _ in range(_SUBL):
                pltpu.make_async_copy(ob.at[pl.ds(0, 2 * _TW)],
                                      out_hbm.at[pl.ds(0, 2 * _TW)],
                                      so).wait()

        def transpose_and_store(pair, rw, ob, so):
            def q_body(q, carry):
                r_base = q * _LANES
                o_base = (lax.rem(q, _SUBL) * 2 * _TW
                          + lax.div(q, _SUBL) * _TW)
                for g in range(_LANES // 16):
                    rws = rw.at[pl.ds(r_base + 16 * g, 16), :]
                    for e in range(d):
                        vals = plsc.load_gather(rws, [iota, e_vecs[e]])
                        ob[pl.ds(o_base + e * _LANES + 16 * g, 16)] = vals
                return carry

            lax.fori_loop(0, 2 * _SUBL, q_body, 0)

            tile = 2 * pair
            lt = tile // bt_tiles
            bt = tile - lt * bt_tiles
            for k in range(_SUBL):
                off = (((lt * _SUBL + k) * bt_tiles) + bt) * _TW
                pltpu.async_copy(ob.at[pl.ds(k * 2 * _TW, 2 * _TW)],
                                 out_hbm.at[pl.ds(off, 2 * _TW)], so)

        start_idx(p0, xb0, si0)
        wait_idx(xb0, si0)
        start_gather(xb0, rw0, sg0)
        start_idx(p0 + 1, xb1, si1)

        def body(p, carry):
            t_a = 2 * p
            t_b = 2 * p + 1

            wait_idx(xb1, si1)
            start_gather(xb1, rw1, sg1)

            @pl.when(p >= 1)
            def _():
                wait_out(ob1, so1)

            wait_gather(xb0, rw0, sg0)

            @pl.when(t_a + 2 < pairs_per_w)
            def _():
                start_idx(p0 + t_a + 2, xb0, si0)

            @pl.when(p >= 1)
            def _():
                wait_out(ob0, so0)

            transpose_and_store(p0 + t_a, rw0, ob0, so0)

            @pl.when(t_b + 2 < pairs_per_w)
            def _():
                wait_idx(xb0, si0)
                start_gather(xb0, rw0, sg0)

            wait_gather(xb1, rw1, sg1)

            @pl.when(t_b + 2 < pairs_per_w)
            def _():
                start_idx(p0 + t_b + 2, xb1, si1)

            transpose_and_store(p0 + t_b, rw1, ob1, so1)
            return carry

        lax.fori_loop(0, pairs_per_w // 2, body, 0)

        wait_out(ob0, so0)
        wait_out(ob1, so1)

    return emb(table, xq)


def kernel(x, table):
    b, l = x.shape
    v, d = table.shape
    assert l % _SUBL == 0 and b % _LANES == 0 and d == _SUBL
    lt_tiles = l // _SUBL
    bt_tiles = b // _LANES
    n_tiles = lt_tiles * bt_tiles

    info = plsc.get_sparse_core_info()
    num_workers = info.num_cores * info.num_subcores
    tiles_per_w = n_tiles // num_workers
    assert n_tiles % num_workers == 0 and tiles_per_w % 2 == 0

    xq = (x.T.reshape(lt_tiles, _SUBL, bt_tiles, _LANES)
          .transpose(0, 2, 1, 3).reshape(b * l))

    tbl_flat = _table_rowmajor(table.T, info.num_cores)
    tbl_rm = tbl_flat.reshape(tbl_flat.shape[0] // d, d)

    out_flat = _emb_lookup(tbl_rm, xq, tiles_per_w, bt_tiles,
                           info.num_cores)

    out = (out_flat.reshape(lt_tiles * _SUBL, bt_tiles, _SUBL, _LANES)
           .transpose(1, 3, 0, 2).reshape(b, l * d))
    return out


# speedup vs baseline: 190.1487x; 1.4325x over previous
import functools

import jax
import jax.numpy as jnp
from jax import lax
from jax.experimental import pallas as pl
from jax.experimental.pallas import tpu as pltpu
from jax.experimental.pallas import tpu_sc as plsc

_LANES = 128
_SUBL = 8
_TW = _LANES * _SUBL


def _table_rowmajor(table_t, num_cores):
    d, v = table_t.shape
    n_full = v // _LANES
    v_tail = v - n_full * _LANES
    bt_sz = 4
    bw = bt_sz * _LANES
    n_batch = n_full // bt_sz
    assert n_full % bt_sz == 0
    num_workers = num_cores * 16
    n_pairs = (pl.cdiv(n_batch, num_workers) + 1) // 2
    mesh = plsc.VectorSubcoreMesh(core_axis_name="c", subcore_axis_name="s")

    @functools.partial(
        pl.kernel,
        mesh=mesh,
        out_type=jax.ShapeDtypeStruct((v * d,), jnp.float32),
        compiler_params=pltpu.CompilerParams(needs_layout_passes=False),
        scratch_types=[
            pltpu.VMEM((d, bw), jnp.float32),
            pltpu.VMEM((d, bw), jnp.float32),
            pltpu.VMEM((bw * d,), jnp.float32),
            pltpu.VMEM((bw * d,), jnp.float32),
            pltpu.SemaphoreType.DMA,
            pltpu.SemaphoreType.DMA,
            pltpu.SemaphoreType.DMA,
            pltpu.SemaphoreType.DMA,
        ],
    )
    def tr(tbl_hbm, out_hbm, tb0, tb1, ob0, ob1, si0, si1, so0, so1):
        wid = lax.axis_index("s") * num_cores + lax.axis_index("c")

        iota = lax.iota(jnp.int32, 16)
        e_vec = lax.rem(iota, d)
        q_vec = lax.div(iota, d)

        def start_in(b, tb, si):
            pltpu.async_copy(tbl_hbm.at[:, pl.ds(b * bw, bw)], tb, si)

        def wait_in(tb, si):
            pltpu.make_async_copy(tbl_hbm.at[:, pl.ds(0, bw)], tb,
                                  si).wait()

        def wait_out(ob, so):
            pltpu.make_async_copy(ob, out_hbm.at[pl.ds(0, bw * d)],
                                  so).wait()

        def transpose_store(b, tb, ob, so):
            def q_body(q, carry):
                j_vec = q_vec + 2 * q
                vals = plsc.load_gather(tb, [e_vec, j_vec])
                ob[pl.ds(16 * q, 16)] = vals
                return carry

            lax.fori_loop(0, (bw * d) // 16, q_body, 0, unroll=8)
            pltpu.async_copy(ob, out_hbm.at[pl.ds(b * bw * d, bw * d)], so)

        def slot(i, b, tb, ob, si, so):
            @pl.when(b < n_batch)
            def _():
                wait_in(tb, si)

                @pl.when(i >= 2)
                def _():
                    wait_out(ob, so)

                transpose_store(b, tb, ob, so)
                b_next = b + 2 * num_workers

                @pl.when(b_next < n_batch)
                def _():
                    start_in(b_next, tb, si)

        @pl.when(wid < n_batch)
        def _():
            start_in(wid, tb0, si0)

        @pl.when(wid + num_workers < n_batch)
        def _():
            start_in(wid + num_workers, tb1, si1)

        def body(p, carry):
            slot(2 * p, wid + 2 * p * num_workers, tb0, ob0, si0, so0)
            slot(2 * p + 1, wid + (2 * p + 1) * num_workers, tb1, ob1,
                 si1, so1)
            return carry

        lax.fori_loop(0, n_pairs, body, 0)

        @pl.when(wid < n_batch)
        def _():
            wait_out(ob0, so0)

        @pl.when(wid + num_workers < n_batch)
        def _():
            wait_out(ob1, so1)

        if v_tail:
            @pl.when(wid == 0)
            def _():
                for e in range(d):
                    pltpu.sync_copy(
                        tbl_hbm.at[e, pl.ds(n_full * _LANES, v_tail)],
                        tb0.at[e, pl.ds(0, v_tail)])

                def q_body(q, carry):
                    j_vec = q_vec + 2 * q
                    vals = plsc.load_gather(tb0, [e_vec, j_vec])
                    ob0[pl.ds(16 * q, 16)] = vals
                    return carry

                lax.fori_loop(0, (v_tail * d) // 16, q_body, 0, unroll=8)
                pltpu.sync_copy(ob0.at[pl.ds(0, v_tail * d)],
                                out_hbm.at[pl.ds(n_full * _TW, v_tail * d)])

    return tr(table_t)


def _emb_lookup(table, xq, tiles_per_w, bt_tiles, num_cores):
    v, d = table.shape
    n_out = xq.shape[0] * d
    pairs_per_w = tiles_per_w // 2
    pw = 2 * _TW
    mesh = plsc.VectorSubcoreMesh(core_axis_name="c", subcore_axis_name="s")

    @functools.partial(
        pl.kernel,
        mesh=mesh,
        out_type=jax.ShapeDtypeStruct((n_out,), jnp.float32),
        compiler_params=pltpu.CompilerParams(
            use_tc_tiling_on_sc=False, needs_layout_passes=False),
        scratch_types=[
            pltpu.VMEM((pw,), jnp.int32),
            pltpu.VMEM((pw,), jnp.int32),
            pltpu.VMEM((pw, d), jnp.float32),
            pltpu.VMEM((pw, d), jnp.float32),
            pltpu.VMEM((pw * d,), jnp.float32),
            pltpu.VMEM((pw * d,), jnp.float32),
            pltpu.SemaphoreType.DMA,
            pltpu.SemaphoreType.DMA,
            pltpu.SemaphoreType.DMA,
            pltpu.SemaphoreType.DMA,
            pltpu.SemaphoreType.DMA,
            pltpu.SemaphoreType.DMA,
        ],
    )
    def emb(table_hbm, xq_hbm, out_hbm, xb0, xb1, rw0, rw1, ob0, ob1,
            si0, si1, sg0, sg1, so0, so1):
        wid = lax.axis_index("s") * num_cores + lax.axis_index("c")
        p0 = wid * pairs_per_w

        iota = lax.iota(jnp.int32, 16)
        e_vecs = [jnp.full((16,), e, jnp.int32) for e in range(d)]

        def start_idx(pair, xb, si):
            pltpu.async_copy(xq_hbm.at[pl.ds(pair * pw, pw)], xb, si)

        def wait_idx(xb, si):
            pltpu.make_async_copy(xq_hbm.at[pl.ds(0, pw)], xb, si).wait()

        def start_gather(xb, rw, sg):
            pltpu.async_copy(table_hbm.at[xb], rw, sg)

        def wait_gather(xb, rw, sg):
            pltpu.make_async_copy(table_hbm.at[xb], rw, sg).wait()

        def wait_out(ob, so):
            for _ in range(_SUBL):
                pltpu.make_async_copy(ob.at[pl.ds(0, 2 * _TW)],
                                      out_hbm.at[pl.ds(0, 2 * _TW)],
                                      so).wait()

        def transpose_and_store(pair, rw, ob, so):
            def q_body(q, carry):
                r_base = q * _LANES
                o_base = (lax.rem(q, _SUBL) * 2 * _TW
                          + lax.div(q, _SUBL) * _TW)
                for g in range(_LANES // 16):
                    rws = rw.at[pl.ds(r_base + 16 * g, 16), :]
                    vals = [plsc.load_gather(rws, [iota, e_vecs[e]])
                            for e in range(d)]
                    for e in range(d):
                        ob[pl.ds(o_base + e * _LANES + 16 * g, 16)] = vals[e]
                return carry

            lax.fori_loop(0, 2 * _SUBL, q_body, 0)

            tile = 2 * pair
            lt = tile // bt_tiles
            bt = tile - lt * bt_tiles
            for k in range(_SUBL):
                off = (((lt * _SUBL + k) * bt_tiles) + bt) * _TW
                pltpu.async_copy(ob.at[pl.ds(k * 2 * _TW, 2 * _TW)],
                                 out_hbm.at[pl.ds(off, 2 * _TW)], so)

        start_idx(p0, xb0, si0)
        wait_idx(xb0, si0)
        start_gather(xb0, rw0, sg0)
        start_idx(p0 + 1, xb1, si1)

        def body(p, carry):
            t_a = 2 * p
            t_b = 2 * p + 1

            wait_idx(xb1, si1)
            start_gather(xb1, rw1, sg1)

            @pl.when(p >= 1)
            def _():
                wait_out(ob1, so1)

            wait_gather(xb0, rw0, sg0)

            @pl.when(t_a + 2 < pairs_per_w)
            def _():
                start_idx(p0 + t_a + 2, xb0, si0)

            @pl.when(p >= 1)
            def _():
                wait_out(ob0, so0)

            transpose_and_store(p0 + t_a, rw0, ob0, so0)

            @pl.when(t_b + 2 < pairs_per_w)
            def _():
                wait_idx(xb0, si0)
                start_gather(xb0, rw0, sg0)

            wait_gather(xb1, rw1, sg1)

            @pl.when(t_b + 2 < pairs_per_w)
            def _():
                start_idx(p0 + t_b + 2, xb1, si1)

            transpose_and_store(p0 + t_b, rw1, ob1, so1)
            return carry

        lax.fori_loop(0, pairs_per_w // 2, body, 0)

        wait_out(ob0, so0)
        wait_out(ob1, so1)

    return emb(table, xq)


def kernel(x, table):
    b, l = x.shape
    v, d = table.shape
    assert l % _SUBL == 0 and b % _LANES == 0 and d == _SUBL
    lt_tiles = l // _SUBL
    bt_tiles = b // _LANES
    n_tiles = lt_tiles * bt_tiles

    info = plsc.get_sparse_core_info()
    num_workers = info.num_cores * info.num_subcores
    tiles_per_w = n_tiles // num_workers
    assert n_tiles % num_workers == 0 and tiles_per_w % 2 == 0

    xq = (x.T.reshape(lt_tiles, _SUBL, bt_tiles, _LANES)
          .transpose(0, 2, 1, 3).reshape(b * l))

    tbl_flat = _table_rowmajor(table.T, info.num_cores)
    tbl_rm = tbl_flat.reshape(tbl_flat.shape[0] // d, d)

    out_flat = _emb_lookup(tbl_rm, xq, tiles_per_w, bt_tiles,
                           info.num_cores)

    out = (out_flat.reshape(lt_tiles * _SUBL, bt_tiles, _SUBL, _LANES)
           .transpose(1, 3, 0, 2).reshape(b, l * d))
    return out


# speedup vs baseline: 241.8661x; 1.2720x over previous
import functools

import jax
import jax.numpy as jnp
from jax import lax
from jax.experimental import pallas as pl
from jax.experimental.pallas import tpu as pltpu
from jax.experimental.pallas import tpu_sc as plsc

_LANES = 128
_SUBL = 8
_TW = _LANES * _SUBL


def _table_rowmajor(table_t, num_cores):
    d, v = table_t.shape
    n_full = v // _LANES
    v_tail = v - n_full * _LANES
    bt_sz = 4
    bw = bt_sz * _LANES
    n_batch = n_full // bt_sz
    assert n_full % bt_sz == 0
    num_workers = num_cores * 16
    n_pairs = (pl.cdiv(n_batch, num_workers) + 1) // 2
    mesh = plsc.VectorSubcoreMesh(core_axis_name="c", subcore_axis_name="s")

    @functools.partial(
        pl.kernel,
        mesh=mesh,
        out_type=jax.ShapeDtypeStruct((v * d,), jnp.float32),
        compiler_params=pltpu.CompilerParams(needs_layout_passes=False),
        scratch_types=[
            pltpu.VMEM((d, bw), jnp.float32),
            pltpu.VMEM((d, bw), jnp.float32),
            pltpu.VMEM((bw * d,), jnp.float32),
            pltpu.VMEM((bw * d,), jnp.float32),
            pltpu.SemaphoreType.DMA,
            pltpu.SemaphoreType.DMA,
            pltpu.SemaphoreType.DMA,
            pltpu.SemaphoreType.DMA,
        ],
    )
    def tr(tbl_hbm, out_hbm, tb0, tb1, ob0, ob1, si0, si1, so0, so1):
        wid = lax.axis_index("s") * num_cores + lax.axis_index("c")

        iota = lax.iota(jnp.int32, 16)
        e_vec = lax.rem(iota, d)
        q_vec = lax.div(iota, d)

        def start_in(b, tb, si):
            pltpu.async_copy(tbl_hbm.at[:, pl.ds(b * bw, bw)], tb, si)

        def wait_in(tb, si):
            pltpu.make_async_copy(tbl_hbm.at[:, pl.ds(0, bw)], tb,
                                  si).wait()

        def wait_out(ob, so):
            pltpu.make_async_copy(ob, out_hbm.at[pl.ds(0, bw * d)],
                                  so).wait()

        def transpose_store(b, tb, ob, so):
            def q_body(q, carry):
                vals = [plsc.load_gather(tb, [e_vec, q_vec + 2 * (8 * q + k)])
                        for k in range(8)]
                for k in range(8):
                    ob[pl.ds(16 * (8 * q + k), 16)] = vals[k]
                return carry

            lax.fori_loop(0, (bw * d) // 128, q_body, 0)
            pltpu.async_copy(ob, out_hbm.at[pl.ds(b * bw * d, bw * d)], so)

        def slot(i, b, tb, ob, si, so):
            @pl.when(b < n_batch)
            def _():
                wait_in(tb, si)

                @pl.when(i >= 2)
                def _():
                    wait_out(ob, so)

                transpose_store(b, tb, ob, so)
                b_next = b + 2 * num_workers

                @pl.when(b_next < n_batch)
                def _():
                    start_in(b_next, tb, si)

        @pl.when(wid < n_batch)
        def _():
            start_in(wid, tb0, si0)

        @pl.when(wid + num_workers < n_batch)
        def _():
            start_in(wid + num_workers, tb1, si1)

        def body(p, carry):
            slot(2 * p, wid + 2 * p * num_workers, tb0, ob0, si0, so0)
            slot(2 * p + 1, wid + (2 * p + 1) * num_workers, tb1, ob1,
                 si1, so1)
            return carry

        lax.fori_loop(0, n_pairs, body, 0)

        @pl.when(wid < n_batch)
        def _():
            wait_out(ob0, so0)

        @pl.when(wid + num_workers < n_batch)
        def _():
            wait_out(ob1, so1)

        if v_tail:
            @pl.when(wid == 0)
            def _():
                for e in range(d):
                    pltpu.sync_copy(
                        tbl_hbm.at[e, pl.ds(n_full * _LANES, v_tail)],
                        tb0.at[e, pl.ds(0, v_tail)])

                def q_body(q, carry):
                    vals = [plsc.load_gather(
                        tb0, [e_vec, q_vec + 2 * (8 * q + k)])
                        for k in range(8)]
                    for k in range(8):
                        ob0[pl.ds(16 * (8 * q + k), 16)] = vals[k]
                    return carry

                lax.fori_loop(0, (v_tail * d) // 128, q_body, 0)
                pltpu.sync_copy(ob0.at[pl.ds(0, v_tail * d)],
                                out_hbm.at[pl.ds(n_full * _TW, v_tail * d)])

    return tr(table_t)


def _emb_lookup(table, xq, tiles_per_w, bt_tiles, num_cores):
    v, d = table.shape
    n_out = xq.shape[0] * d
    pairs_per_w = tiles_per_w // 2
    pw = 2 * _TW
    mesh = plsc.VectorSubcoreMesh(core_axis_name="c", subcore_axis_name="s")

    @functools.partial(
        pl.kernel,
        mesh=mesh,
        out_type=jax.ShapeDtypeStruct((n_out,), jnp.float32),
        compiler_params=pltpu.CompilerParams(
            use_tc_tiling_on_sc=False, needs_layout_passes=False),
        scratch_types=[
            pltpu.VMEM((pw,), jnp.int32),
            pltpu.VMEM((pw,), jnp.int32),
            pltpu.VMEM((pw, d), jnp.float32),
            pltpu.VMEM((pw, d), jnp.float32),
            pltpu.VMEM((pw * d,), jnp.float32),
            pltpu.VMEM((pw * d,), jnp.float32),
            pltpu.SemaphoreType.DMA,
            pltpu.SemaphoreType.DMA,
            pltpu.SemaphoreType.DMA,
            pltpu.SemaphoreType.DMA,
            pltpu.SemaphoreType.DMA,
            pltpu.SemaphoreType.DMA,
        ],
    )
    def emb(table_hbm, xq_hbm, out_hbm, xb0, xb1, rw0, rw1, ob0, ob1,
            si0, si1, sg0, sg1, so0, so1):
        wid = lax.axis_index("s") * num_cores + lax.axis_index("c")
        p0 = wid * pairs_per_w

        iota = lax.iota(jnp.int32, 16)
        e_vecs = [jnp.full((16,), e, jnp.int32) for e in range(d)]

        def start_idx(pair, xb, si):
            pltpu.async_copy(xq_hbm.at[pl.ds(pair * pw, pw)], xb, si)

        def wait_idx(xb, si):
            pltpu.make_async_copy(xq_hbm.at[pl.ds(0, pw)], xb, si).wait()

        def start_gather(xb, rw, sg):
            pltpu.async_copy(table_hbm.at[xb], rw, sg)

        def wait_gather(xb, rw, sg):
            pltpu.make_async_copy(table_hbm.at[xb], rw, sg).wait()

        def wait_out(ob, so):
            for _ in range(_SUBL):
                pltpu.make_async_copy(ob.at[pl.ds(0, 2 * _TW)],
                                      out_hbm.at[pl.ds(0, 2 * _TW)],
                                      so).wait()

        def transpose_and_store(pair, rw, ob, so):
            def q_body(q, carry):
                r_base = q * _LANES
                o_base = (lax.rem(q, _SUBL) * 2 * _TW
                          + lax.div(q, _SUBL) * _TW)
                for g in range(_LANES // 16):
                    rws = rw.at[pl.ds(r_base + 16 * g, 16), :]
                    vals = [plsc.load_gather(rws, [iota, e_vecs[e]])
                            for e in range(d)]
                    for e in range(d):
                        ob[pl.ds(o_base + e * _LANES + 16 * g, 16)] = vals[e]
                return carry

            lax.fori_loop(0, 2 * _SUBL, q_body, 0)

            tile = 2 * pair
            lt = tile // bt_tiles
            bt = tile - lt * bt_tiles
            for k in range(_SUBL):
                off = (((lt * _SUBL + k) * bt_tiles) + bt) * _TW
                pltpu.async_copy(ob.at[pl.ds(k * 2 * _TW, 2 * _TW)],
                                 out_hbm.at[pl.ds(off, 2 * _TW)], so)

        start_idx(p0, xb0, si0)
        wait_idx(xb0, si0)
        start_gather(xb0, rw0, sg0)
        start_idx(p0 + 1, xb1, si1)

        def body(p, carry):
            t_a = 2 * p
            t_b = 2 * p + 1

            wait_idx(xb1, si1)
            start_gather(xb1, rw1, sg1)

            @pl.when(p >= 1)
            def _():
                wait_out(ob1, so1)

            wait_gather(xb0, rw0, sg0)

            @pl.when(t_a + 2 < pairs_per_w)
            def _():
                start_idx(p0 + t_a + 2, xb0, si0)

            @pl.when(p >= 1)
            def _():
                wait_out(ob0, so0)

            transpose_and_store(p0 + t_a, rw0, ob0, so0)

            @pl.when(t_b + 2 < pairs_per_w)
            def _():
                wait_idx(xb0, si0)
                start_gather(xb0, rw0, sg0)

            wait_gather(xb1, rw1, sg1)

            @pl.when(t_b + 2 < pairs_per_w)
            def _():
                start_idx(p0 + t_b + 2, xb1, si1)

            transpose_and_store(p0 + t_b, rw1, ob1, so1)
            return carry

        lax.fori_loop(0, pairs_per_w // 2, body, 0)

        wait_out(ob0, so0)
        wait_out(ob1, so1)

    return emb(table, xq)


def kernel(x, table):
    b, l = x.shape
    v, d = table.shape
    assert l % _SUBL == 0 and b % _LANES == 0 and d == _SUBL
    lt_tiles = l // _SUBL
    bt_tiles = b // _LANES
    n_tiles = lt_tiles * bt_tiles

    info = plsc.get_sparse_core_info()
    num_workers = info.num_cores * info.num_subcores
    tiles_per_w = n_tiles // num_workers
    assert n_tiles % num_workers == 0 and tiles_per_w % 2 == 0

    xq = (x.T.reshape(lt_tiles, _SUBL, bt_tiles, _LANES)
          .transpose(0, 2, 1, 3).reshape(b * l))

    tbl_flat = _table_rowmajor(table.T, info.num_cores)
    tbl_rm = tbl_flat.reshape(tbl_flat.shape[0] // d, d)

    out_flat = _emb_lookup(tbl_rm, xq, tiles_per_w, bt_tiles,
                           info.num_cores)

    out = (out_flat.reshape(lt_tiles * _SUBL, bt_tiles, _SUBL, _LANES)
           .transpose(1, 3, 0, 2).reshape(b, l * d))
    return out
